# baseline (speedup 1.0000x reference)
import jax
import jax.numpy as jnp
import numpy as np
from jax.experimental import pallas as pl

_PACK = 8
_BLK = 2000


def _mlp_kernel(x_ref, w1_ref, b1_ref, w2_ref, b2_ref, o_ref):
    x = x_ref[...]
    h = jnp.dot(x, w1_ref[...], preferred_element_type=jnp.float32) + b1_ref[...]
    h = 0.5 * h * (1.0 + jax.lax.erf(h * (1.0 / np.sqrt(2.0))))
    o_ref[...] = jnp.dot(h, w2_ref[...], preferred_element_type=jnp.float32) + b2_ref[...]


def kernel(node_features, edge_features, edge_index, node_tiers,
           Wq1, Wk1, Wv1, We1, Wo1, Wq2, Wk2, Wv2, We2, Wo2,
           Wc1, bc1, Wc2, bc2):
    E, DE = edge_features.shape
    C = Wc2.shape[1]
    R = E // _PACK
    din = _PACK * DE
    dout = _PACK * C

    x = edge_features.reshape(R, din)
    eye = jnp.eye(_PACK, dtype=jnp.float32)
    w1 = jnp.kron(eye, Wc1.astype(jnp.float32))
    w2 = jnp.kron(eye, Wc2.astype(jnp.float32))
    b1 = jnp.tile(bc1.astype(jnp.float32), _PACK)[None, :]
    b2 = jnp.tile(bc2.astype(jnp.float32), _PACK)[None, :]

    out = pl.pallas_call(
        _mlp_kernel,
        grid=(R // _BLK,),
        in_specs=[
            pl.BlockSpec((_BLK, din), lambda i: (i, 0)),
            pl.BlockSpec((din, din), lambda i: (0, 0)),
            pl.BlockSpec((1, din), lambda i: (0, 0)),
            pl.BlockSpec((din, dout), lambda i: (0, 0)),
            pl.BlockSpec((1, dout), lambda i: (0, 0)),
        ],
        out_specs=pl.BlockSpec((_BLK, dout), lambda i: (i, 0)),
        out_shape=jax.ShapeDtypeStruct((R, dout), jnp.float32),
    )(x, w1, b1, w2, b2)
    return out.reshape(E, C)


# speedup vs baseline: 1.6742x; 1.6742x over previous
import functools

import jax
import jax.numpy as jnp
import numpy as np
from jax.experimental import pallas as pl
from jax.experimental.pallas import tpu as pltpu

_PACK = 8
_BLK = 2000


def _mlp_kernel(x_ref, w1_ref, b1_ref, w2_ref, b2_ref, o_ref):
    c = o_ref.shape[2]
    x8 = x_ref[...]
    xp = jnp.concatenate([x8[s] for s in range(_PACK)], axis=1)
    h = jnp.dot(xp, w1_ref[...], preferred_element_type=jnp.float32) + b1_ref[...]
    h = 0.5 * h * (1.0 + jax.lax.erf(h * np.float32(1.0 / np.sqrt(2.0))))
    o = jnp.dot(h, w2_ref[...], preferred_element_type=jnp.float32) + b2_ref[...]
    o_ref[...] = jnp.stack([o[:, c * s:c * (s + 1)] for s in range(_PACK)], axis=0)


def kernel(node_features, edge_features, edge_index, node_tiers,
           Wq1, Wk1, Wv1, We1, Wo1, Wq2, Wk2, Wv2, We2, Wo2,
           Wc1, bc1, Wc2, bc2):
    E, DE = edge_features.shape
    C = Wc2.shape[1]
    e8 = E // _PACK
    din = _PACK * DE
    dout = _PACK * C
    nblk = e8 // _BLK

    eye = jnp.eye(_PACK, dtype=jnp.float32)
    w1 = jnp.kron(eye, Wc1.astype(jnp.float32))
    w2 = jnp.kron(eye, Wc2.astype(jnp.float32))
    b1 = jnp.tile(bc1.astype(jnp.float32), _PACK)[None, :]
    b2 = jnp.tile(bc2.astype(jnp.float32), _PACK)[None, :]

    x3 = edge_features.reshape(_PACK, e8, DE)

    out3 = pl.pallas_call(
        _mlp_kernel,
        grid=(nblk,),
        in_specs=[
            pl.BlockSpec((_PACK, _BLK, DE), lambda i: (0, i, 0)),
            pl.BlockSpec((din, din), lambda i: (0, 0)),
            pl.BlockSpec((1, din), lambda i: (0, 0)),
            pl.BlockSpec((din, dout), lambda i: (0, 0)),
            pl.BlockSpec((1, dout), lambda i: (0, 0)),
        ],
        out_specs=pl.BlockSpec((_PACK, _BLK, C), lambda i: (0, i, 0)),
        out_shape=jax.ShapeDtypeStruct((_PACK, e8, C), jnp.float32),
        compiler_params=pltpu.CompilerParams(
            dimension_semantics=("arbitrary",),
        ),
    )(x3, w1, b1, w2, b2)
    return out3.reshape(E, C)
